# Initial kernel scaffold; baseline (speedup 1.0000x reference)
import jax
import jax.numpy as jnp
from jax.experimental import pallas as pl


def kernel(visual_feat, pred_logits, pair_idx, W_sub, b_sub, W_obj, b_obj, W_cls, b_cls):
    raise NotImplementedError("write your pallas kernel here")


# speedup vs baseline: 6.0124x; 6.0124x over previous
import functools

import jax
import jax.numpy as jnp
from jax import lax
from jax.experimental import pallas as pl
from jax.experimental.pallas import tpu as pltpu
from jax.experimental.pallas import tpu_sc as plsc

N_OBJ = 2048
NUM_CLS = 151
HIDDEN = 256
P = 131072
NN = N_OBJ * N_OBJ

NTILES = 16
PAIRS_PER_TILE = P // NTILES
CHUNKS = PAIRS_PER_TILE // 16
ZROWS = NN // NTILES
ZBUF = 16384
NZDMA = ZROWS // ZBUF
SCAT_B = 128
SCAT_N = PAIRS_PER_TILE // SCAT_B


def _tc_vals_body(lg_ref, ws_ref, bs_ref, wo_ref, bo_ref, wcs_ref, wco_ref,
                  bc_ref, sval_ref, oval_ref):
    x = lg_ref[...]
    m = jnp.max(x, axis=1, keepdims=True)
    e = jnp.exp(x - m)
    p = e / jnp.sum(e, axis=1, keepdims=True)
    hs = jnp.maximum(
        jnp.dot(p, ws_ref[...], preferred_element_type=jnp.float32) + bs_ref[...], 0.0)
    ho = jnp.maximum(
        jnp.dot(p, wo_ref[...], preferred_element_type=jnp.float32) + bo_ref[...], 0.0)
    sval_ref[...] = jnp.sum(hs * wcs_ref[...], axis=1, keepdims=True) + bc_ref[0, 0]
    oval_ref[...] = jnp.sum(ho * wco_ref[...], axis=1, keepdims=True)


_tc_vals = pl.pallas_call(
    _tc_vals_body,
    out_shape=(jax.ShapeDtypeStruct((N_OBJ, 1), jnp.float32),
               jax.ShapeDtypeStruct((N_OBJ, 1), jnp.float32)),
)


def _sc_body(pairs_hbm, sval_hbm, oval_hbm, logits_hbm, mat_hbm,
             pair_v, stab_v, otab_v, logit_v, score_v, fidx_v, zero_v, dump_v,
             zsem, ssem):
    w = lax.axis_index("s")

    def zfill(k, carry):
        zero_v[pl.ds(k * 16, 16)] = jnp.zeros((16,), jnp.float32)
        return carry
    lax.fori_loop(0, ZBUF // 16, zfill, 0)

    zbase = w * ZROWS
    zcopies = [
        pltpu.async_copy(zero_v, mat_hbm.at[pl.ds(zbase + z * ZBUF, ZBUF)], zsem)
        for z in range(NZDMA)
    ]

    pltpu.sync_copy(sval_hbm, stab_v)
    pltpu.sync_copy(oval_hbm, otab_v)
    pltpu.sync_copy(
        pairs_hbm.at[pl.ds(w * 2 * PAIRS_PER_TILE, 2 * PAIRS_PER_TILE)], pair_v)

    def body(m, carry):
        lane = lax.iota(jnp.int32, 16)
        b2 = m * 32
        ii = plsc.load_gather(pair_v, [b2 + 2 * lane])
        jj = plsc.load_gather(pair_v, [b2 + 2 * lane + 1])
        sv = plsc.load_gather(stab_v, [ii])
        ov = plsc.load_gather(otab_v, [jj])
        lg = sv + ov
        logit_v[pl.ds(m * 16, 16)] = lg
        sc = 1.0 / (1.0 + jnp.exp(-lg))
        r = m // 8
        o = (m % 8) * 16
        score_v[r, pl.ds(o, 16)] = sc
        fidx_v[r, pl.ds(o, 16)] = ii * N_OBJ + jj
        return carry
    lax.fori_loop(0, CHUNKS, body, 0)

    pltpu.sync_copy(logit_v, logits_hbm.at[pl.ds(w * PAIRS_PER_TILE, PAIRS_PER_TILE)])

    for d in zcopies:
        d.wait()
    plsc.subcore_barrier()

    def sbody(cc, carry):
        pltpu.async_copy(score_v.at[cc], mat_hbm.at[fidx_v.at[cc]], ssem)
        return carry
    lax.fori_loop(0, SCAT_N, sbody, 0)
    pltpu.make_async_copy(logits_hbm.at[pl.ds(0, SCAT_N * SCAT_B)], dump_v, ssem).wait()


_sc_scatter = functools.partial(
    pl.kernel,
    out_type=(jax.ShapeDtypeStruct((P,), jnp.float32),
              jax.ShapeDtypeStruct((NN,), jnp.float32)),
    mesh=plsc.VectorSubcoreMesh(
        core_axis_name="c", subcore_axis_name="s", num_cores=1),
    compiler_params=pltpu.CompilerParams(needs_layout_passes=False),
    scratch_types=(
        pltpu.VMEM((2 * PAIRS_PER_TILE,), jnp.int32),
        pltpu.VMEM((N_OBJ,), jnp.float32),
        pltpu.VMEM((N_OBJ,), jnp.float32),
        pltpu.VMEM((PAIRS_PER_TILE,), jnp.float32),
        pltpu.VMEM((SCAT_N, SCAT_B), jnp.float32),
        pltpu.VMEM((SCAT_N, SCAT_B), jnp.int32),
        pltpu.VMEM((ZBUF,), jnp.float32),
        pltpu.VMEM((SCAT_N * SCAT_B,), jnp.float32),
        pltpu.SemaphoreType.DMA,
        pltpu.SemaphoreType.DMA,
    ),
)(_sc_body)


def kernel(visual_feat, pred_logits, pair_idx, W_sub, b_sub, W_obj, b_obj,
           W_cls, b_cls):
    del visual_feat
    ws_cls = W_cls[:HIDDEN].reshape(1, HIDDEN)
    wo_cls = W_cls[HIDDEN:].reshape(1, HIDDEN)
    sval, oval = _tc_vals(pred_logits, W_sub, b_sub.reshape(1, HIDDEN),
                          W_obj, b_obj.reshape(1, HIDDEN),
                          ws_cls, wo_cls, b_cls.reshape(1, 1))
    logits, mat = _sc_scatter(pair_idx.reshape(-1), sval.reshape(-1),
                              oval.reshape(-1))
    return logits, mat.reshape(N_OBJ, N_OBJ)
